# baseline (speedup 1.0000x reference)
import functools

import jax
import jax.numpy as jnp
from jax import lax
from jax.experimental import pallas as pl
from jax.experimental.pallas import tpu as pltpu
from jax.experimental.pallas import tpu_sc as plsc

CHOICE_ALPHA = 0.001
COUNT_PENALTY_GAMMA = 0.01
VIGILANCE = 0.75

B, K, D, XD = 128, 1024, 512, 256
L = 16
NC, NS = 2, 16
NW = NC * NS
RPW = B // NW
KSC = 128
KTC = K - KSC
KC = 64
NG = KSC // KC
CD = D // L
NQ = 4
QD = D // NQ
QC = QD // L

NEG_INF = float("-inf")

_GATHER_DN = lax.GatherDimensionNumbers(
    offset_dims=(), collapsed_slice_dims=(0,), start_index_map=(0,))


def _lane_iota():
  return lax.iota(jnp.int32, L)


def _lanes(v, idx):
  return lax.gather(v, idx[:, None], _GATHER_DN, slice_sizes=(1,),
                    mode=lax.GatherScatterMode.PROMISE_IN_BOUNDS)


def _allsum(v):
  for s in (1, 2, 4, 8):
    v = v + _lanes(v, jnp.bitwise_xor(_lane_iota(), s))
  return v


def _splat(x, dtype=jnp.float32):
  return jnp.full((L,), x, dtype=dtype)


def _tree_sum(vals):
  vals = list(vals)
  while len(vals) > 1:
    nxt = [vals[i] + vals[i + 1] for i in range(0, len(vals) - 1, 2)]
    if len(vals) % 2:
      nxt.append(vals[-1])
    vals = nxt
  return vals[0]


def _vm_splat(ref, base, lane):
  grp = ref[pl.ds(base, L)]
  return _lanes(grp, _splat(lane, jnp.int32))


def _fuzzy_art_body(x_hbm, counts_hbm, comm_hbm, t_hbm,
                    cv_hbm, w_hbm, b_hbm, n_hbm, cs_hbm,
                    xv, coded, tbuf, penal, negmask, tsbuf, accbuf, cvrows,
                    stage_i, stage_f, counts_v, comm_v, sem0, sem1):
  wid = lax.axis_index("s") * NC + lax.axis_index("c")
  row0 = wid * RPW

  pltpu.sync_copy(x_hbm.at[pl.ds(row0, RPW)], xv)
  pltpu.sync_copy(counts_hbm, counts_v)
  pltpu.sync_copy(comm_hbm, comm_v)

  sems = (sem0, sem1)
  for p in range(2):
    pltpu.make_async_copy(
        t_hbm.at[pl.ds(p * KC, KC)], tbuf.at[p], sems[p]).start()

  def scal_body(i, _):
    cnt = counts_v[pl.ds(i * L, L)].astype(jnp.float32)
    penal[pl.ds(i * L, L)] = CHOICE_ALPHA + COUNT_PENALTY_GAMMA * cnt
    com = comm_v[pl.ds(i * L, L)]
    negmask[pl.ds(i * L, L)] = jnp.where(com > 0, jnp.float32(0.0), NEG_INF)
    return 0
  lax.fori_loop(0, KSC // L, scal_body, 0)

  csum = []
  for b in range(RPW):
    acc = jnp.zeros((L,), jnp.float32)
    for i in range(XD // L):
      xvv = xv[b, pl.ds(i * L, L)]
      cvv = 1.0 - xvv
      coded[b, pl.ds(i * L, L)] = xvv
      coded[b, pl.ds(XD + i * L, L)] = cvv
      acc = acc + xvv + cvv
    csum.append(_allsum(acc))

  def make_q_body(q, g, p):
    creg = [[coded[b, pl.ds(q * QD + c * L, L)] for c in range(QC)]
            for b in range(RPW)]

    def part_body(kl, carry):
      tvs = [tbuf[p, kl, pl.ds(q * QD + c * L, L)] for c in range(QC)]
      tpart = _tree_sum(tvs)
      if q == 0:
        tsbuf[pl.ds(kl * L, L)] = tpart
      else:
        plsc.addupdate(tsbuf.at[pl.ds(kl * L, L)], tpart)
      for b in range(RPW):
        part = _tree_sum(
            [jnp.minimum(creg[b][c], tvs[c]) for c in range(QC)])
        if q == 0:
          accbuf[b, pl.ds(kl * L, L)] = part
        else:
          plsc.addupdate(accbuf.at[b, pl.ds(kl * L, L)], part)
      return carry

    def last_body(kl, carry):
      carry = list(carry)
      kglob = g * KC + kl
      kbase = g * KC + (kl // L) * L
      klane = kl % L
      tvs = [tbuf[p, kl, pl.ds(q * QD + c * L, L)] for c in range(QC)]
      ts = tsbuf[pl.ds(kl * L, L)] + _tree_sum(tvs)
      den = _allsum(ts) + _vm_splat(penal, kbase, klane)
      rden = 1.0 / den
      neg = _vm_splat(negmask, kbase, klane)
      kvec = _splat(kglob, jnp.int32)
      lsel = _lane_iota() == klane
      for b in range(RPW):
        best, bi, bn, cvgrp = carry[4 * b:4 * b + 4]
        part = _tree_sum(
            [jnp.minimum(creg[b][c], tvs[c]) for c in range(QC)])
        acc = accbuf[b, pl.ds(kl * L, L)] + part
        num = _allsum(acc)
        cv = num * rden + neg
        cvgrp = jnp.where(lsel, cv, cvgrp)
        m = cv > best
        best = jnp.where(m, cv, best)
        bi = jnp.where(m, kvec, bi)
        bn = jnp.where(m, num, bn)
        carry[4 * b:4 * b + 4] = [best, bi, bn, cvgrp]

      @pl.when(klane == L - 1)
      def _():
        for b in range(RPW):
          cvrows[pl.ds(b * KSC + kbase, L)] = carry[4 * b + 3]

      return tuple(carry)

    return part_body if q < NQ - 1 else last_body

  zf = jnp.zeros((L,), jnp.float32)
  zi = jnp.zeros((L,), jnp.int32)
  init = []
  for b in range(RPW):
    init += [jnp.full((L,), NEG_INF), zi, zf]

  def outer_body(gi, carry):
    carry = list(carry)
    for p in range(2):
      g = gi * 2 + p
      pltpu.make_async_copy(
          t_hbm.at[pl.ds(g * KC, KC)], tbuf.at[p], sems[p]).wait()
      for q in range(NQ - 1):
        lax.fori_loop(0, KC, make_q_body(q, g, p), 0, unroll=2)
      st = list(lax.fori_loop(
          0, KC, make_q_body(NQ - 1, g, p),
          tuple(x for b in range(RPW)
                for x in (carry[3 * b], carry[3 * b + 1],
                          carry[3 * b + 2], zf)),
          unroll=2))
      for b in range(RPW):
        carry[3 * b:3 * b + 3] = st[4 * b:4 * b + 3]
      g2 = g + 2

      @pl.when(g2 < NG)
      def _():
        pltpu.make_async_copy(
            t_hbm.at[pl.ds(g2 * KC, KC)], tbuf.at[p], sems[p]).start()
    return tuple(carry)

  fin = lax.fori_loop(0, NG // 2, outer_body, tuple(init))

  wvec = zi
  bvec = zf
  nvec = zf
  csvec = zf
  for b in range(RPW):
    best, bi, bn = fin[3 * b:3 * b + 3]
    sel = _lane_iota() == b
    wvec = jnp.where(sel, bi, wvec)
    bvec = jnp.where(sel, best, bvec)
    nvec = jnp.where(sel, bn, nvec)
    csvec = jnp.where(sel, csum[b], csvec)

  stage_i[...] = wvec
  pltpu.sync_copy(stage_i, w_hbm.at[wid])
  stage_f[...] = bvec
  pltpu.sync_copy(stage_f, b_hbm.at[wid])
  stage_f[...] = nvec
  pltpu.sync_copy(stage_f, n_hbm.at[wid])
  stage_f[...] = csvec
  pltpu.sync_copy(stage_f, cs_hbm.at[wid])
  pltpu.sync_copy(cvrows, cv_hbm.at[pl.ds(row0 * KSC, RPW * KSC)])


_fuzzy_art = functools.partial(
    pl.kernel,
    out_type=(
        jax.ShapeDtypeStruct((B * KSC,), jnp.float32),
        jax.ShapeDtypeStruct((NW, L), jnp.int32),
        jax.ShapeDtypeStruct((NW, L), jnp.float32),
        jax.ShapeDtypeStruct((NW, L), jnp.float32),
        jax.ShapeDtypeStruct((NW, L), jnp.float32),
    ),
    mesh=plsc.VectorSubcoreMesh(core_axis_name="c", subcore_axis_name="s"),
    scratch_types=[
        pltpu.VMEM((RPW, XD), jnp.float32),
        pltpu.VMEM((RPW, D), jnp.float32),
        pltpu.VMEM((2, KC, D), jnp.float32),
        pltpu.VMEM((KSC,), jnp.float32),
        pltpu.VMEM((KSC,), jnp.float32),
        pltpu.VMEM((KC * L,), jnp.float32),
        pltpu.VMEM((RPW, KC * L), jnp.float32),
        pltpu.VMEM((RPW * KSC,), jnp.float32),
        pltpu.VMEM((L,), jnp.int32),
        pltpu.VMEM((L,), jnp.float32),
        pltpu.VMEM((KSC,), jnp.int32),
        pltpu.VMEM((KSC,), jnp.int32),
        pltpu.SemaphoreType.DMA,
        pltpu.SemaphoreType.DMA,
    ],
)(_fuzzy_art_body)


def _tc_body(x_ref, t_ref, counts_ref, comm_ref,
             cvt_ref, best_ref, bi_ref, bn_ref, num_scr):
  xb = x_ref[...]
  coded = jnp.concatenate([xb, 1.0 - xb], axis=-1)
  ones = jnp.ones((1, D), jnp.float32)

  def k_body(i, carry):
    base = pl.multiple_of(i * 8, 8)
    ttile = t_ref[pl.ds(base, 8), :]
    rows = []
    for r in range(8):
      trow = ttile[r, :]
      m = jnp.minimum(coded, trow[None, :])
      rows.append(lax.dot_general(
          ones, m, (((1,), (1,)), ((), ())),
          preferred_element_type=jnp.float32))
    num_scr[pl.ds(base, 8), :] = jnp.concatenate(rows, axis=0)
    return carry

  lax.fori_loop(0, KTC // 8, k_body, 0)

  best = jnp.full((B,), NEG_INF, jnp.float32)
  bi = jnp.zeros((B,), jnp.int32)
  bn = jnp.zeros((B,), jnp.float32)
  for j in range(KTC // 128):
    sl = pl.ds(j * 128, 128)
    blk = num_scr[sl, :]
    ts = lax.dot_general(ones, t_ref[sl, :], (((1,), (1,)), ((), ())),
                         preferred_element_type=jnp.float32)[0]
    cnt = counts_ref[sl].astype(jnp.float32)
    den = CHOICE_ALPHA + ts + COUNT_PENALTY_GAMMA * cnt
    neg = jnp.where(comm_ref[sl] > 0, 0.0, NEG_INF)
    cvb = blk * (1.0 / den)[:, None] + neg[:, None]
    cvt_ref[sl, :] = cvb
    bmax = jnp.max(cvb, axis=0)
    bidx = jnp.argmax(cvb, axis=0).astype(jnp.int32) + j * 128
    bnum = jnp.max(jnp.where(cvb == bmax[None, :], blk, NEG_INF), axis=0)
    m = bmax > best
    best = jnp.where(m, bmax, best)
    bi = jnp.where(m, bidx, bi)
    bn = jnp.where(m, bnum, bn)
  best_ref[...] = best
  bi_ref[...] = bi
  bn_ref[...] = bn


_fuzzy_art_tc = pl.pallas_call(
    _tc_body,
    out_shape=(
        jax.ShapeDtypeStruct((KTC, B), jnp.float32),
        jax.ShapeDtypeStruct((B,), jnp.float32),
        jax.ShapeDtypeStruct((B,), jnp.int32),
        jax.ShapeDtypeStruct((B,), jnp.float32),
    ),
    scratch_shapes=[pltpu.VMEM((KTC, B), jnp.float32)],
)


def kernel(x, templates, category_counts, committed):
  counts_i = category_counts.astype(jnp.int32)
  comm_i = committed.astype(jnp.int32)
  cvt, b_tc, i_tc, n_tc = _fuzzy_art_tc(
      x, templates[KSC:], counts_i[KSC:], comm_i[KSC:])
  cvs, w_sc, b_sc, n_sc, cs = _fuzzy_art(
      x, counts_i[:KSC], comm_i[:KSC], templates[:KSC])

  w_sc = w_sc[:, :RPW].reshape(B)
  b_sc = b_sc[:, :RPW].reshape(B)
  n_sc = n_sc[:, :RPW].reshape(B)
  csum = cs[:, :RPW].reshape(B)

  use_tc = b_tc > b_sc
  winners = jnp.where(use_tc, i_tc + KSC, w_sc)
  bn = jnp.where(use_tc, n_tc, n_sc)
  match = bn / (csum + 1e-10)
  resonance = match >= VIGILANCE
  cv = jnp.concatenate([cvs.reshape(B, KSC), cvt.T], axis=1)
  return cv, winners, match, resonance


# speedup vs baseline: 1.3130x; 1.3130x over previous
import functools

import jax
import jax.numpy as jnp
from jax import lax
from jax.experimental import pallas as pl
from jax.experimental.pallas import tpu as pltpu
from jax.experimental.pallas import tpu_sc as plsc

CHOICE_ALPHA = 0.001
COUNT_PENALTY_GAMMA = 0.01
VIGILANCE = 0.75

B, K, D, XD = 128, 1024, 512, 256
L = 16
NC, NS = 2, 16
NW = NC * NS
RPW = B // NW
KSC = 128
KTC = K - KSC
KC = 64
NG = KSC // KC
CD = D // L
NQ = 4
QD = D // NQ
QC = QD // L

NEG_INF = float("-inf")

_GATHER_DN = lax.GatherDimensionNumbers(
    offset_dims=(), collapsed_slice_dims=(0,), start_index_map=(0,))


def _lane_iota():
  return lax.iota(jnp.int32, L)


def _lanes(v, idx):
  return lax.gather(v, idx[:, None], _GATHER_DN, slice_sizes=(1,),
                    mode=lax.GatherScatterMode.PROMISE_IN_BOUNDS)


def _allsum(v):
  for s in (1, 2, 4, 8):
    v = v + _lanes(v, jnp.bitwise_xor(_lane_iota(), s))
  return v


def _splat(x, dtype=jnp.float32):
  return jnp.full((L,), x, dtype=dtype)


def _tree_sum(vals):
  vals = list(vals)
  while len(vals) > 1:
    nxt = [vals[i] + vals[i + 1] for i in range(0, len(vals) - 1, 2)]
    if len(vals) % 2:
      nxt.append(vals[-1])
    vals = nxt
  return vals[0]


def _vm_splat(ref, base, lane):
  grp = ref[pl.ds(base, L)]
  return _lanes(grp, _splat(lane, jnp.int32))


def _fuzzy_art_body(x_hbm, counts_hbm, comm_hbm, t_hbm,
                    cv_hbm, w_hbm, b_hbm, n_hbm, cs_hbm,
                    xv, coded, tbuf, penal, negmask, tsbuf, accbuf, cvrows,
                    stage_i, stage_f, counts_v, comm_v, sem0, sem1):
  wid = lax.axis_index("s") * NC + lax.axis_index("c")
  row0 = wid * RPW

  pltpu.sync_copy(x_hbm.at[pl.ds(row0, RPW)], xv)
  pltpu.sync_copy(counts_hbm, counts_v)
  pltpu.sync_copy(comm_hbm, comm_v)

  sems = (sem0, sem1)
  for p in range(2):
    pltpu.make_async_copy(
        t_hbm.at[pl.ds(p * KC, KC)], tbuf.at[p], sems[p]).start()

  def scal_body(i, _):
    cnt = counts_v[pl.ds(i * L, L)].astype(jnp.float32)
    penal[pl.ds(i * L, L)] = CHOICE_ALPHA + COUNT_PENALTY_GAMMA * cnt
    com = comm_v[pl.ds(i * L, L)]
    negmask[pl.ds(i * L, L)] = jnp.where(com > 0, jnp.float32(0.0), NEG_INF)
    return 0
  lax.fori_loop(0, KSC // L, scal_body, 0)

  csum = []
  for b in range(RPW):
    acc = jnp.zeros((L,), jnp.float32)
    for i in range(XD // L):
      xvv = xv[b, pl.ds(i * L, L)]
      cvv = 1.0 - xvv
      coded[b, pl.ds(i * L, L)] = xvv
      coded[b, pl.ds(XD + i * L, L)] = cvv
      acc = acc + xvv + cvv
    csum.append(_allsum(acc))

  def make_q_body(q, g, p):
    creg = [[coded[b, pl.ds(q * QD + c * L, L)] for c in range(QC)]
            for b in range(RPW)]

    def part_body(kl, carry):
      tvs = [tbuf[p, kl, pl.ds(q * QD + c * L, L)] for c in range(QC)]
      tpart = _tree_sum(tvs)
      if q == 0:
        tsbuf[pl.ds(kl * L, L)] = tpart
      else:
        plsc.addupdate(tsbuf.at[pl.ds(kl * L, L)], tpart)
      for b in range(RPW):
        part = _tree_sum(
            [jnp.minimum(creg[b][c], tvs[c]) for c in range(QC)])
        if q == 0:
          accbuf[b, pl.ds(kl * L, L)] = part
        else:
          plsc.addupdate(accbuf.at[b, pl.ds(kl * L, L)], part)
      return carry

    def last_body(kl, carry):
      carry = list(carry)
      kglob = g * KC + kl
      kbase = g * KC + (kl // L) * L
      klane = kl % L
      tvs = [tbuf[p, kl, pl.ds(q * QD + c * L, L)] for c in range(QC)]
      ts = tsbuf[pl.ds(kl * L, L)] + _tree_sum(tvs)
      den = _allsum(ts) + _vm_splat(penal, kbase, klane)
      rden = 1.0 / den
      neg = _vm_splat(negmask, kbase, klane)
      kvec = _splat(kglob, jnp.int32)
      lsel = _lane_iota() == klane
      for b in range(RPW):
        best, bi, bn, cvgrp = carry[4 * b:4 * b + 4]
        part = _tree_sum(
            [jnp.minimum(creg[b][c], tvs[c]) for c in range(QC)])
        acc = accbuf[b, pl.ds(kl * L, L)] + part
        num = _allsum(acc)
        cv = num * rden + neg
        cvgrp = jnp.where(lsel, cv, cvgrp)
        m = cv > best
        best = jnp.where(m, cv, best)
        bi = jnp.where(m, kvec, bi)
        bn = jnp.where(m, num, bn)
        carry[4 * b:4 * b + 4] = [best, bi, bn, cvgrp]

      @pl.when(klane == L - 1)
      def _():
        for b in range(RPW):
          cvrows[pl.ds(b * KSC + kbase, L)] = carry[4 * b + 3]

      return tuple(carry)

    return part_body if q < NQ - 1 else last_body

  zf = jnp.zeros((L,), jnp.float32)
  zi = jnp.zeros((L,), jnp.int32)
  init = []
  for b in range(RPW):
    init += [jnp.full((L,), NEG_INF), zi, zf]

  def outer_body(gi, carry):
    carry = list(carry)
    for p in range(2):
      g = gi * 2 + p
      pltpu.make_async_copy(
          t_hbm.at[pl.ds(g * KC, KC)], tbuf.at[p], sems[p]).wait()
      for q in range(NQ - 1):
        lax.fori_loop(0, KC, make_q_body(q, g, p), 0, unroll=2)
      st = list(lax.fori_loop(
          0, KC, make_q_body(NQ - 1, g, p),
          tuple(x for b in range(RPW)
                for x in (carry[3 * b], carry[3 * b + 1],
                          carry[3 * b + 2], zf)),
          unroll=2))
      for b in range(RPW):
        carry[3 * b:3 * b + 3] = st[4 * b:4 * b + 3]
      g2 = g + 2

      @pl.when(g2 < NG)
      def _():
        pltpu.make_async_copy(
            t_hbm.at[pl.ds(g2 * KC, KC)], tbuf.at[p], sems[p]).start()
    return tuple(carry)

  fin = lax.fori_loop(0, NG // 2, outer_body, tuple(init))

  wvec = zi
  bvec = zf
  nvec = zf
  csvec = zf
  for b in range(RPW):
    best, bi, bn = fin[3 * b:3 * b + 3]
    sel = _lane_iota() == b
    wvec = jnp.where(sel, bi, wvec)
    bvec = jnp.where(sel, best, bvec)
    nvec = jnp.where(sel, bn, nvec)
    csvec = jnp.where(sel, csum[b], csvec)

  stage_i[...] = wvec
  pltpu.sync_copy(stage_i, w_hbm.at[wid])
  stage_f[...] = bvec
  pltpu.sync_copy(stage_f, b_hbm.at[wid])
  stage_f[...] = nvec
  pltpu.sync_copy(stage_f, n_hbm.at[wid])
  stage_f[...] = csvec
  pltpu.sync_copy(stage_f, cs_hbm.at[wid])
  pltpu.sync_copy(cvrows, cv_hbm.at[pl.ds(row0 * KSC, RPW * KSC)])


_fuzzy_art = functools.partial(
    pl.kernel,
    out_type=(
        jax.ShapeDtypeStruct((B * KSC,), jnp.float32),
        jax.ShapeDtypeStruct((NW, L), jnp.int32),
        jax.ShapeDtypeStruct((NW, L), jnp.float32),
        jax.ShapeDtypeStruct((NW, L), jnp.float32),
        jax.ShapeDtypeStruct((NW, L), jnp.float32),
    ),
    mesh=plsc.VectorSubcoreMesh(core_axis_name="c", subcore_axis_name="s"),
    scratch_types=[
        pltpu.VMEM((RPW, XD), jnp.float32),
        pltpu.VMEM((RPW, D), jnp.float32),
        pltpu.VMEM((2, KC, D), jnp.float32),
        pltpu.VMEM((KSC,), jnp.float32),
        pltpu.VMEM((KSC,), jnp.float32),
        pltpu.VMEM((KC * L,), jnp.float32),
        pltpu.VMEM((RPW, KC * L), jnp.float32),
        pltpu.VMEM((RPW * KSC,), jnp.float32),
        pltpu.VMEM((L,), jnp.int32),
        pltpu.VMEM((L,), jnp.float32),
        pltpu.VMEM((KSC,), jnp.int32),
        pltpu.VMEM((KSC,), jnp.int32),
        pltpu.SemaphoreType.DMA,
        pltpu.SemaphoreType.DMA,
    ],
)(_fuzzy_art_body)


def _tc_body(x_ref, t_ref, counts_ref, comm_ref,
             cvt_ref, best_ref, bi_ref, bn_ref, num_scr):
  xb = x_ref[...]
  coded = jnp.concatenate([xb, 1.0 - xb], axis=-1)

  def k_body(i, carry):
    base = pl.multiple_of(i * 8, 8)
    ttile = t_ref[pl.ds(base, 8), :]
    rows = []
    for r in range(8):
      trow = ttile[r, :]
      m = jnp.minimum(coded, trow[None, :])
      rows.append(jnp.sum(m, axis=1))
    num_scr[pl.ds(base, 8), :] = jnp.stack(rows, axis=0)
    return carry

  lax.fori_loop(0, KTC // 8, k_body, 0)

  best = jnp.full((B,), NEG_INF, jnp.float32)
  bi = jnp.zeros((B,), jnp.int32)
  bn = jnp.zeros((B,), jnp.float32)
  for j in range(KTC // 128):
    sl = pl.ds(j * 128, 128)
    blk = num_scr[sl, :]
    ts = jnp.sum(t_ref[sl, :], axis=1)
    cnt = counts_ref[sl].astype(jnp.float32)
    den = CHOICE_ALPHA + ts + COUNT_PENALTY_GAMMA * cnt
    neg = jnp.where(comm_ref[sl] > 0, 0.0, NEG_INF)
    cvb = blk * (1.0 / den)[:, None] + neg[:, None]
    cvt_ref[sl, :] = cvb
    bmax = jnp.max(cvb, axis=0)
    bidx = jnp.argmax(cvb, axis=0).astype(jnp.int32) + j * 128
    bnum = jnp.max(jnp.where(cvb == bmax[None, :], blk, NEG_INF), axis=0)
    m = bmax > best
    best = jnp.where(m, bmax, best)
    bi = jnp.where(m, bidx, bi)
    bn = jnp.where(m, bnum, bn)
  best_ref[...] = best
  bi_ref[...] = bi
  bn_ref[...] = bn


_fuzzy_art_tc = pl.pallas_call(
    _tc_body,
    out_shape=(
        jax.ShapeDtypeStruct((KTC, B), jnp.float32),
        jax.ShapeDtypeStruct((B,), jnp.float32),
        jax.ShapeDtypeStruct((B,), jnp.int32),
        jax.ShapeDtypeStruct((B,), jnp.float32),
    ),
    scratch_shapes=[pltpu.VMEM((KTC, B), jnp.float32)],
)


def kernel(x, templates, category_counts, committed):
  counts_i = category_counts.astype(jnp.int32)
  comm_i = committed.astype(jnp.int32)
  cvt, b_tc, i_tc, n_tc = _fuzzy_art_tc(
      x, templates[KSC:], counts_i[KSC:], comm_i[KSC:])
  cvs, w_sc, b_sc, n_sc, cs = _fuzzy_art(
      x, counts_i[:KSC], comm_i[:KSC], templates[:KSC])

  w_sc = w_sc[:, :RPW].reshape(B)
  b_sc = b_sc[:, :RPW].reshape(B)
  n_sc = n_sc[:, :RPW].reshape(B)
  csum = cs[:, :RPW].reshape(B)

  use_tc = b_tc > b_sc
  winners = jnp.where(use_tc, i_tc + KSC, w_sc)
  bn = jnp.where(use_tc, n_tc, n_sc)
  match = bn / (csum + 1e-10)
  resonance = match >= VIGILANCE
  cv = jnp.concatenate([cvs.reshape(B, KSC), cvt.T], axis=1)
  return cv, winners, match, resonance
